# Initial kernel scaffold; baseline (speedup 1.0000x reference)
import jax
import jax.numpy as jnp
from jax.experimental import pallas as pl


def kernel(xyz1, xyz2, points1, points2, W1, b1, g1, be1, W2, b2, g2, be2):
    raise NotImplementedError("write your pallas kernel here")


# speedup vs baseline: 15.3168x; 15.3168x over previous
import functools

import jax
import jax.numpy as jnp
from jax.experimental import pallas as pl

B, N, S = 8, 4096, 1024
C1, C2 = 256, 512
CO = 256
TN = 512
_CNT = float(B * N)
_HI = jax.lax.Precision.HIGHEST


def _k0(p2t_ref, w1bt_ref, wp2_ref):
    wp2_ref[0] = jnp.dot(p2t_ref[0], w1bt_ref[...], precision=_HI,
                         preferred_element_type=jnp.float32)


def _ka(x1_ref, x2_ref, p1_ref, wp2_ref, w1at_ref, b1_ref,
        h1_ref, s_ref, ss_ref):
    b = pl.program_id(0)
    nt = pl.program_id(1)

    x1 = x1_ref[0]
    x2 = x2_ref[0]
    d0 = jnp.dot(x1.astype(jnp.bfloat16), x2.astype(jnp.bfloat16),
                 preferred_element_type=jnp.float32)
    n1 = jnp.sum(x1 * x1, axis=1, keepdims=True)
    n2 = jnp.sum(x2 * x2, axis=0, keepdims=True)
    dist = d0 * (-2.0) + n1 + n2

    bits = jax.lax.bitcast_convert_type(dist, jnp.int32)
    sgn = jnp.bitwise_and(jnp.right_shift(bits, 31), jnp.int32(0x7FFFFFFF))
    kb = jnp.bitwise_xor(bits, sgn)
    lane = jax.lax.broadcasted_iota(jnp.int32, dist.shape, 1)
    keys0 = jnp.bitwise_or(jnp.bitwise_and(kb, jnp.int32(-1024)), lane)

    keys = keys0
    big = jnp.int32(2**31 - 1)
    bigf = jnp.float32(3.0e38)
    ms, dv = [], []
    for _ in range(3):
        m = jnp.min(keys, axis=1, keepdims=True)
        ms.append(m)
        eq = keys0 == m
        dv.append(jnp.min(jnp.where(eq, dist, bigf), axis=1, keepdims=True))
        keys = jnp.where(eq, big, keys)
    w = [1.0 / (d + 1e-8) for d in dv]
    wtot = w[0] + w[1] + w[2]
    wn = [wi / wtot for wi in w]
    wmat = (jnp.where(keys0 == ms[0], wn[0], 0.0)
            + jnp.where(keys0 == ms[1], wn[1], 0.0)
            + jnp.where(keys0 == ms[2], wn[2], 0.0))

    h = (jnp.dot(wmat, wp2_ref[0], precision=_HI,
                 preferred_element_type=jnp.float32)
         + jnp.dot(p1_ref[0], w1at_ref[...], precision=_HI,
                   preferred_element_type=jnp.float32)
         + b1_ref[...])
    h1_ref[0] = h

    @pl.when((b == 0) & (nt == 0))
    def _():
        s_ref[...] = jnp.zeros_like(s_ref)
        ss_ref[...] = jnp.zeros_like(ss_ref)

    hr = h.reshape(TN // 8, 8, CO)
    s_ref[...] += jnp.sum(hr, axis=0)
    ss_ref[...] += jnp.sum(hr * hr, axis=0)


def _kb(h1_ref, s_ref, ss_ref, g1_ref, be1_ref, w2t_ref, b2_ref,
        h2_ref, s2_ref, ss2_ref):
    b = pl.program_id(0)
    nt = pl.program_id(1)
    mean = jnp.sum(s_ref[...], axis=0, keepdims=True) * (1.0 / _CNT)
    var = (jnp.sum(ss_ref[...], axis=0, keepdims=True) * (1.0 / _CNT)
           - mean * mean)
    scale = g1_ref[...] * jax.lax.rsqrt(var + 1e-5)
    shift = be1_ref[...] - mean * scale
    a = jnp.maximum(h1_ref[0] * scale + shift, 0.0)
    h2 = jnp.dot(a, w2t_ref[...], precision=_HI,
                 preferred_element_type=jnp.float32) + b2_ref[...]
    h2_ref[0] = h2

    @pl.when((b == 0) & (nt == 0))
    def _():
        s2_ref[...] = jnp.zeros_like(s2_ref)
        ss2_ref[...] = jnp.zeros_like(ss2_ref)

    h2r = h2.reshape(TN // 8, 8, CO)
    s2_ref[...] += jnp.sum(h2r, axis=0)
    ss2_ref[...] += jnp.sum(h2r * h2r, axis=0)


def _kc(h2_ref, s_ref, ss_ref, g2_ref, be2_ref, o_ref):
    mean = jnp.sum(s_ref[...], axis=0, keepdims=True) * (1.0 / _CNT)
    var = (jnp.sum(ss_ref[...], axis=0, keepdims=True) * (1.0 / _CNT)
           - mean * mean)
    scale = g2_ref[...] * jax.lax.rsqrt(var + 1e-5)
    shift = be2_ref[...] - mean * scale
    o_ref[0] = jnp.maximum(h2_ref[0] * scale + shift, 0.0)


def kernel(xyz1, xyz2, points1, points2, W1, b1, g1, be1, W2, b2, g2, be2):
    f32 = jnp.float32
    nt = N // TN

    x1t = jnp.concatenate(
        [jnp.transpose(xyz1, (0, 2, 1)), jnp.zeros((B, N, 5), f32)], axis=-1)
    x2p = jnp.concatenate([xyz2, jnp.zeros((B, 5, S), f32)], axis=1)
    p1t = jnp.transpose(points1, (0, 2, 1))
    p2t = jnp.transpose(points2, (0, 2, 1))
    w1at = jnp.transpose(W1[:, :C1])
    w1bt = jnp.transpose(W1[:, C1:])
    w2t = jnp.transpose(W2)
    b1r, g1r, be1r = b1[None, :], g1[None, :], be1[None, :]
    b2r, g2r, be2r = b2[None, :], g2[None, :], be2[None, :]

    wp2 = pl.pallas_call(
        _k0,
        grid=(B,),
        in_specs=[
            pl.BlockSpec((1, S, C2), lambda b: (b, 0, 0)),
            pl.BlockSpec((C2, CO), lambda b: (0, 0)),
        ],
        out_specs=pl.BlockSpec((1, S, CO), lambda b: (b, 0, 0)),
        out_shape=jax.ShapeDtypeStruct((B, S, CO), f32),
    )(p2t, w1bt)

    h1, s1, ss1 = pl.pallas_call(
        _ka,
        grid=(B, nt),
        in_specs=[
            pl.BlockSpec((1, TN, 8), lambda b, i: (b, i, 0)),
            pl.BlockSpec((1, 8, S), lambda b, i: (b, 0, 0)),
            pl.BlockSpec((1, TN, C1), lambda b, i: (b, i, 0)),
            pl.BlockSpec((1, S, CO), lambda b, i: (b, 0, 0)),
            pl.BlockSpec((C1, CO), lambda b, i: (0, 0)),
            pl.BlockSpec((1, CO), lambda b, i: (0, 0)),
        ],
        out_specs=[
            pl.BlockSpec((1, TN, CO), lambda b, i: (b, i, 0)),
            pl.BlockSpec((8, CO), lambda b, i: (0, 0)),
            pl.BlockSpec((8, CO), lambda b, i: (0, 0)),
        ],
        out_shape=[
            jax.ShapeDtypeStruct((B, N, CO), f32),
            jax.ShapeDtypeStruct((8, CO), f32),
            jax.ShapeDtypeStruct((8, CO), f32),
        ],
    )(x1t, x2p, p1t, wp2, w1at, b1r)

    h2, s2, ss2 = pl.pallas_call(
        _kb,
        grid=(B, nt),
        in_specs=[
            pl.BlockSpec((1, TN, CO), lambda b, i: (b, i, 0)),
            pl.BlockSpec((8, CO), lambda b, i: (0, 0)),
            pl.BlockSpec((8, CO), lambda b, i: (0, 0)),
            pl.BlockSpec((1, CO), lambda b, i: (0, 0)),
            pl.BlockSpec((1, CO), lambda b, i: (0, 0)),
            pl.BlockSpec((CO, CO), lambda b, i: (0, 0)),
            pl.BlockSpec((1, CO), lambda b, i: (0, 0)),
        ],
        out_specs=[
            pl.BlockSpec((1, TN, CO), lambda b, i: (b, i, 0)),
            pl.BlockSpec((8, CO), lambda b, i: (0, 0)),
            pl.BlockSpec((8, CO), lambda b, i: (0, 0)),
        ],
        out_shape=[
            jax.ShapeDtypeStruct((B, N, CO), f32),
            jax.ShapeDtypeStruct((8, CO), f32),
            jax.ShapeDtypeStruct((8, CO), f32),
        ],
    )(h1, s1, ss1, g1r, be1r, w2t, b2r)

    out_nm = pl.pallas_call(
        _kc,
        grid=(B, nt),
        in_specs=[
            pl.BlockSpec((1, TN, CO), lambda b, i: (b, i, 0)),
            pl.BlockSpec((8, CO), lambda b, i: (0, 0)),
            pl.BlockSpec((8, CO), lambda b, i: (0, 0)),
            pl.BlockSpec((1, CO), lambda b, i: (0, 0)),
            pl.BlockSpec((1, CO), lambda b, i: (0, 0)),
        ],
        out_specs=pl.BlockSpec((1, TN, CO), lambda b, i: (b, i, 0)),
        out_shape=jax.ShapeDtypeStruct((B, N, CO), f32),
    )(h2, s2, ss2, g2r, be2r)

    return jnp.transpose(out_nm, (0, 2, 1))


# speedup vs baseline: 21.1810x; 1.3829x over previous
import functools

import jax
import jax.numpy as jnp
from jax.experimental import pallas as pl

B, N, S = 8, 4096, 1024
C1, C2 = 256, 512
CO = 256
TN = 512
_CNT = float(B * N)
_HI = jax.lax.Precision.HIGHEST


def _k0(p2t_ref, w1bt_ref, wp2_ref):
    wp2_ref[0] = jnp.dot(p2t_ref[0], w1bt_ref[...], precision=_HI,
                         preferred_element_type=jnp.float32)


def _ka(x1_ref, x2_ref, p1_ref, wp2_ref, w1at_ref, b1_ref,
        h1_ref, s_ref, ss_ref):
    b = pl.program_id(0)
    nt = pl.program_id(1)

    x1 = x1_ref[0]
    x2 = x2_ref[0]
    d0 = jnp.dot(x1.astype(jnp.bfloat16), x2.astype(jnp.bfloat16),
                 preferred_element_type=jnp.float32)
    n1 = jnp.sum(x1 * x1, axis=1, keepdims=True)
    n2 = jnp.sum(x2 * x2, axis=0, keepdims=True)
    dist = d0 * (-2.0) + n1 + n2

    bits = jax.lax.bitcast_convert_type(dist, jnp.int32)
    sgn = jnp.bitwise_and(jnp.right_shift(bits, 31), jnp.int32(0x7FFFFFFF))
    kb = jnp.bitwise_xor(bits, sgn)
    lane = jax.lax.broadcasted_iota(jnp.int32, dist.shape, 1)
    keys0 = jnp.bitwise_or(jnp.bitwise_and(kb, jnp.int32(-1024)), lane)

    keys = keys0
    big = jnp.int32(2**31 - 1)
    bigf = jnp.float32(3.0e38)
    ms, dv = [], []
    for _ in range(3):
        m = jnp.min(keys, axis=1, keepdims=True)
        ms.append(m)
        eq = keys0 == m
        dv.append(jnp.min(jnp.where(eq, dist, bigf), axis=1, keepdims=True))
        keys = jnp.where(eq, big, keys)
    w = [1.0 / (d + 1e-8) for d in dv]
    wtot = w[0] + w[1] + w[2]
    wn = [wi / wtot for wi in w]
    wmat = (jnp.where(keys0 == ms[0], wn[0], 0.0)
            + jnp.where(keys0 == ms[1], wn[1], 0.0)
            + jnp.where(keys0 == ms[2], wn[2], 0.0))

    bf16 = jnp.bfloat16
    h = (jnp.dot(wmat.astype(bf16), wp2_ref[0].astype(bf16),
                 preferred_element_type=jnp.float32)
         + jnp.dot(p1_ref[0].astype(bf16), w1at_ref[...].astype(bf16),
                   preferred_element_type=jnp.float32)
         + b1_ref[...])
    h1_ref[0] = h

    @pl.when((b == 0) & (nt == 0))
    def _():
        s_ref[...] = jnp.zeros_like(s_ref)
        ss_ref[...] = jnp.zeros_like(ss_ref)

    hr = h.reshape(TN // 8, 8, CO)
    s_ref[...] += jnp.sum(hr, axis=0)
    ss_ref[...] += jnp.sum(hr * hr, axis=0)


def _kb(h1_ref, s_ref, ss_ref, g1_ref, be1_ref, w2t_ref, b2_ref,
        h2_ref, s2_ref, ss2_ref):
    b = pl.program_id(0)
    nt = pl.program_id(1)
    mean = jnp.sum(s_ref[...], axis=0, keepdims=True) * (1.0 / _CNT)
    var = (jnp.sum(ss_ref[...], axis=0, keepdims=True) * (1.0 / _CNT)
           - mean * mean)
    scale = g1_ref[...] * jax.lax.rsqrt(var + 1e-5)
    shift = be1_ref[...] - mean * scale
    a = jnp.maximum(h1_ref[0] * scale + shift, 0.0)
    h2 = jnp.dot(a.astype(jnp.bfloat16), w2t_ref[...].astype(jnp.bfloat16),
                 preferred_element_type=jnp.float32) + b2_ref[...]
    h2_ref[0] = h2

    @pl.when((b == 0) & (nt == 0))
    def _():
        s2_ref[...] = jnp.zeros_like(s2_ref)
        ss2_ref[...] = jnp.zeros_like(ss2_ref)

    h2r = h2.reshape(TN // 8, 8, CO)
    s2_ref[...] += jnp.sum(h2r, axis=0)
    ss2_ref[...] += jnp.sum(h2r * h2r, axis=0)


def _kc(h2_ref, s_ref, ss_ref, g2_ref, be2_ref, o_ref):
    mean = jnp.sum(s_ref[...], axis=0, keepdims=True) * (1.0 / _CNT)
    var = (jnp.sum(ss_ref[...], axis=0, keepdims=True) * (1.0 / _CNT)
           - mean * mean)
    scale = g2_ref[...] * jax.lax.rsqrt(var + 1e-5)
    shift = be2_ref[...] - mean * scale
    o_ref[0] = jnp.maximum(h2_ref[0] * scale + shift, 0.0)


def kernel(xyz1, xyz2, points1, points2, W1, b1, g1, be1, W2, b2, g2, be2):
    f32 = jnp.float32
    nt = N // TN

    x1t = jnp.concatenate(
        [jnp.transpose(xyz1, (0, 2, 1)), jnp.zeros((B, N, 5), f32)], axis=-1)
    x2p = jnp.concatenate([xyz2, jnp.zeros((B, 5, S), f32)], axis=1)
    p1t = jnp.transpose(points1, (0, 2, 1))
    p2t = jnp.transpose(points2, (0, 2, 1))
    w1at = jnp.transpose(W1[:, :C1])
    w1bt = jnp.transpose(W1[:, C1:])
    w2t = jnp.transpose(W2)
    b1r, g1r, be1r = b1[None, :], g1[None, :], be1[None, :]
    b2r, g2r, be2r = b2[None, :], g2[None, :], be2[None, :]

    wp2 = pl.pallas_call(
        _k0,
        grid=(B,),
        in_specs=[
            pl.BlockSpec((1, S, C2), lambda b: (b, 0, 0)),
            pl.BlockSpec((C2, CO), lambda b: (0, 0)),
        ],
        out_specs=pl.BlockSpec((1, S, CO), lambda b: (b, 0, 0)),
        out_shape=jax.ShapeDtypeStruct((B, S, CO), f32),
    )(p2t, w1bt)

    h1, s1, ss1 = pl.pallas_call(
        _ka,
        grid=(B, nt),
        in_specs=[
            pl.BlockSpec((1, TN, 8), lambda b, i: (b, i, 0)),
            pl.BlockSpec((1, 8, S), lambda b, i: (b, 0, 0)),
            pl.BlockSpec((1, TN, C1), lambda b, i: (b, i, 0)),
            pl.BlockSpec((1, S, CO), lambda b, i: (b, 0, 0)),
            pl.BlockSpec((C1, CO), lambda b, i: (0, 0)),
            pl.BlockSpec((1, CO), lambda b, i: (0, 0)),
        ],
        out_specs=[
            pl.BlockSpec((1, TN, CO), lambda b, i: (b, i, 0)),
            pl.BlockSpec((8, CO), lambda b, i: (0, 0)),
            pl.BlockSpec((8, CO), lambda b, i: (0, 0)),
        ],
        out_shape=[
            jax.ShapeDtypeStruct((B, N, CO), f32),
            jax.ShapeDtypeStruct((8, CO), f32),
            jax.ShapeDtypeStruct((8, CO), f32),
        ],
    )(x1t, x2p, p1t, wp2, w1at, b1r)

    h2, s2, ss2 = pl.pallas_call(
        _kb,
        grid=(B, nt),
        in_specs=[
            pl.BlockSpec((1, TN, CO), lambda b, i: (b, i, 0)),
            pl.BlockSpec((8, CO), lambda b, i: (0, 0)),
            pl.BlockSpec((8, CO), lambda b, i: (0, 0)),
            pl.BlockSpec((1, CO), lambda b, i: (0, 0)),
            pl.BlockSpec((1, CO), lambda b, i: (0, 0)),
            pl.BlockSpec((CO, CO), lambda b, i: (0, 0)),
            pl.BlockSpec((1, CO), lambda b, i: (0, 0)),
        ],
        out_specs=[
            pl.BlockSpec((1, TN, CO), lambda b, i: (b, i, 0)),
            pl.BlockSpec((8, CO), lambda b, i: (0, 0)),
            pl.BlockSpec((8, CO), lambda b, i: (0, 0)),
        ],
        out_shape=[
            jax.ShapeDtypeStruct((B, N, CO), f32),
            jax.ShapeDtypeStruct((8, CO), f32),
            jax.ShapeDtypeStruct((8, CO), f32),
        ],
    )(h1, s1, ss1, g1r, be1r, w2t, b2r)

    out_nm = pl.pallas_call(
        _kc,
        grid=(B, nt),
        in_specs=[
            pl.BlockSpec((1, TN, CO), lambda b, i: (b, i, 0)),
            pl.BlockSpec((8, CO), lambda b, i: (0, 0)),
            pl.BlockSpec((8, CO), lambda b, i: (0, 0)),
            pl.BlockSpec((1, CO), lambda b, i: (0, 0)),
            pl.BlockSpec((1, CO), lambda b, i: (0, 0)),
        ],
        out_specs=pl.BlockSpec((1, TN, CO), lambda b, i: (b, i, 0)),
        out_shape=jax.ShapeDtypeStruct((B, N, CO), f32),
    )(h2, s2, ss2, g2r, be2r)

    return jnp.transpose(out_nm, (0, 2, 1))
